# baseline (speedup 1.0000x reference)
import jax
import jax.numpy as jnp
from jax import lax
from jax.experimental import pallas as pl
from jax.experimental.pallas import tpu as pltpu
from jax.experimental.pallas import tpu_sc as plsc


_LANES = 16
_NW = 32
_RING = 8
_AHEAD = 4


def _sc_add(x2d, pos_emb, chunk_rows=4):
    R, D = x2d.shape
    S = pos_emb.shape[0]
    B = R // S
    C = chunk_rows
    s_per_w = S // _NW
    n_chunks = s_per_w // C
    n_groups = n_chunks // _RING
    mesh = plsc.VectorSubcoreMesh(core_axis_name="c", subcore_axis_name="s")

    def body(x_hbm, pe_hbm, out_hbm, xbuf, pebuf, load_sems, store_sems):
        wid = lax.axis_index("s") * 2 + lax.axis_index("c")
        s_base = wid * s_per_w

        def load_descs(k, slot):
            s0 = s_base + k * C
            descs = [
                pltpu.make_async_copy(
                    pe_hbm.at[pl.ds(s0, C)], pebuf.at[slot], load_sems.at[slot]
                )
            ]
            for b in range(B):
                descs.append(
                    pltpu.make_async_copy(
                        x_hbm.at[pl.ds(b * S + s0, C)],
                        xbuf.at[slot, b],
                        load_sems.at[slot],
                    )
                )
            return descs

        def store_descs(k, slot):
            s0 = s_base + k * C
            return [
                pltpu.make_async_copy(
                    xbuf.at[slot, b],
                    out_hbm.at[pl.ds(b * S + s0, C)],
                    store_sems.at[slot],
                )
                for b in range(B)
            ]

        def compute(slot):
            def row_add(r, carry):
                for k in range(D // _LANES):
                    sl = pl.ds(k * _LANES, _LANES)
                    pv = pebuf[slot, r, sl]
                    for b in range(B):
                        plsc.addupdate(xbuf.at[slot, b, r, sl], pv)
                return carry

            lax.fori_loop(0, C, row_add, 0, unroll=False)

        slack = _RING - _AHEAD
        for p in range(_AHEAD):
            for d in load_descs(p, p):
                d.start()

        def group(g, carry):
            for j in range(_RING):
                k = g * _RING + j
                slot_ahead = (j + _AHEAD) % _RING

                @pl.when(k >= slack)
                def _():
                    for d in store_descs(k - slack, slot_ahead):
                        d.wait()

                @pl.when(k + _AHEAD < n_chunks)
                def _():
                    for d in load_descs(k + _AHEAD, slot_ahead):
                        d.start()

                for d in load_descs(k, j):
                    d.wait()
                compute(j)
                for d in store_descs(k, j):
                    d.start()
            return carry

        lax.fori_loop(0, n_groups, group, 0, unroll=False)
        for k in range(n_chunks - slack, n_chunks):
            for d in store_descs(k, k % _RING):
                d.wait()

    fn = pl.kernel(
        body,
        out_type=jax.ShapeDtypeStruct((R, D), x2d.dtype),
        mesh=mesh,
        scratch_types=[
            pltpu.VMEM((_RING, B, C, D), jnp.float32),
            pltpu.VMEM((_RING, C, D), jnp.float32),
            pltpu.SemaphoreType.DMA((_RING,)),
            pltpu.SemaphoreType.DMA((_RING,)),
        ],
    )
    return fn(x2d, pos_emb)


def kernel(x, pos_emb):
    B, S, D = x.shape
    x2d = x.reshape(B * S, D)
    out2d = _sc_add(x2d, pos_emb[:S])
    return out2d.reshape(B, S, D)


# speedup vs baseline: 1.0830x; 1.0830x over previous
import jax
import jax.numpy as jnp
from jax import lax
from jax.experimental import pallas as pl
from jax.experimental.pallas import tpu as pltpu
from jax.experimental.pallas import tpu_sc as plsc


_LANES = 16
_NW = 32
_RING = 4
_AHEAD = 1


def _sc_add(x2d, pos_emb, chunk_rows=8):
    R, D = x2d.shape
    S = pos_emb.shape[0]
    B = R // S
    C = chunk_rows
    s_per_w = S // _NW
    n_chunks = s_per_w // C
    n_groups = n_chunks // _RING
    mesh = plsc.VectorSubcoreMesh(core_axis_name="c", subcore_axis_name="s")

    def body(x_hbm, pe_hbm, out_hbm, xbuf, pebuf, load_sems, store_sems):
        wid = lax.axis_index("s") * 2 + lax.axis_index("c")
        s_base = wid * s_per_w

        def load_descs(k, slot):
            s0 = s_base + k * C
            descs = [
                pltpu.make_async_copy(
                    pe_hbm.at[pl.ds(s0, C)], pebuf.at[slot], load_sems.at[slot]
                )
            ]
            for b in range(B):
                descs.append(
                    pltpu.make_async_copy(
                        x_hbm.at[pl.ds(b * S + s0, C)],
                        xbuf.at[slot, b],
                        load_sems.at[slot],
                    )
                )
            return descs

        def store_descs(k, slot):
            s0 = s_base + k * C
            return [
                pltpu.make_async_copy(
                    xbuf.at[slot, b],
                    out_hbm.at[pl.ds(b * S + s0, C)],
                    store_sems.at[slot],
                )
                for b in range(B)
            ]

        def compute(slot):
            def row_add(r, carry):
                for k in range(D // _LANES):
                    sl = pl.ds(k * _LANES, _LANES)
                    pv = pebuf[slot, r, sl]
                    for b in range(B):
                        plsc.addupdate(xbuf.at[slot, b, r, sl], pv)
                return carry

            lax.fori_loop(0, C, row_add, 0, unroll=False)

        slack = _RING - _AHEAD
        for p in range(_AHEAD):
            for d in load_descs(p, p):
                d.start()

        def group(g, carry):
            for j in range(_RING):
                k = g * _RING + j
                slot_ahead = (j + _AHEAD) % _RING

                @pl.when(k >= slack)
                def _():
                    for d in store_descs(k - slack, slot_ahead):
                        d.wait()

                @pl.when(k + _AHEAD < n_chunks)
                def _():
                    for d in load_descs(k + _AHEAD, slot_ahead):
                        d.start()

                for d in load_descs(k, j):
                    d.wait()
                compute(j)
                for d in store_descs(k, j):
                    d.start()
            return carry

        lax.fori_loop(0, n_groups, group, 0, unroll=False)
        for k in range(n_chunks - slack, n_chunks):
            for d in store_descs(k, k % _RING):
                d.wait()

    fn = pl.kernel(
        body,
        out_type=jax.ShapeDtypeStruct((R, D), x2d.dtype),
        mesh=mesh,
        scratch_types=[
            pltpu.VMEM((_RING, B, C, D), jnp.float32),
            pltpu.VMEM((_RING, C, D), jnp.float32),
            pltpu.SemaphoreType.DMA((_RING,)),
            pltpu.SemaphoreType.DMA((_RING,)),
        ],
    )
    return fn(x2d, pos_emb)


def kernel(x, pos_emb):
    B, S, D = x.shape
    x2d = x.reshape(B * S, D)
    out2d = _sc_add(x2d, pos_emb[:S])
    return out2d.reshape(B, S, D)
